# baseline (speedup 1.0000x reference)
import functools

import jax
import jax.numpy as jnp
from jax import lax
from jax.experimental import pallas as pl
from jax.experimental.pallas import tpu as pltpu
from jax.experimental.pallas import tpu_sc as plsc

_B = 16384
_E = 25
_VOCAB = 26
_VP = 27
_PAD_ROW = 26
_NC = 2
_NS = 16
_NW = _NC * _NS
_BPW = _B // _NW
_L = 16
_TABW = _VP * _E * _L


@functools.lru_cache(maxsize=1)
def _build():
    mesh = plsc.VectorSubcoreMesh(
        core_axis_name="c", subcore_axis_name="s",
        num_cores=_NC, num_subcores=_NS,
    )

    @functools.partial(
        pl.kernel,
        out_type=jax.ShapeDtypeStruct((_B * _E,), jnp.float32),
        mesh=mesh,
        scratch_types=[
            pltpu.VMEM((_TABW,), jnp.float32),
            pltpu.VMEM((_BPW,), jnp.int32),
            pltpu.VMEM((_BPW,), jnp.int32),
            pltpu.VMEM((_BPW * _E,), jnp.float32),
            pltpu.SemaphoreType.DMA,
        ],
        compiler_params=pltpu.CompilerParams(needs_layout_passes=False),
    )
    def emb(w_hbm, col_hbm, len_hbm, out_hbm, tab_v, col_v, len_v, rows_v, sem):
        wid = lax.axis_index("s") * _NC + lax.axis_index("c")
        base = wid * _BPW
        cps = [
            pltpu.async_copy(w_hbm, tab_v, sem),
            pltpu.async_copy(col_hbm.at[pl.ds(base, _BPW)], col_v, sem),
            pltpu.async_copy(len_hbm.at[pl.ds(base, _BPW)], len_v, sem),
        ]
        for cp in cps:
            cp.wait()
        lanes = lax.iota(jnp.int32, _L)
        lanes_e = lanes * _E

        @plsc.parallel_loop(0, _BPW, _L, unroll=4)
        def _(off):
            tok = col_v[pl.ds(off, _L)]
            ln = len_v[pl.ds(off, _L)]
            idx = jnp.where(ln > 0, tok, _PAD_ROW)
            lbase = idx * (_E * _L) + lanes
            sbase = lanes_e + off * _E
            for c in range(_E):
                vals = plsc.load_gather(tab_v, [lbase + (c * _L)])
                plsc.store_scatter(rows_v, [sbase + c], vals)

        pltpu.sync_copy(rows_v, out_hbm.at[pl.ds(base * _E, _BPW * _E)])

    return emb


def kernel(input, lengths, weight):
    col = input[:, 0]
    wflat = jnp.concatenate(
        [weight, jnp.zeros((1, _E), jnp.float32)], axis=0
    ).reshape(-1)
    wrep = jnp.repeat(wflat, _L)
    return _build()(wrep, col, lengths).reshape(_B, _E)


# speedup vs baseline: 1.7049x; 1.7049x over previous
import functools

import jax
import jax.numpy as jnp
from jax import lax
from jax.experimental import pallas as pl
from jax.experimental.pallas import tpu as pltpu
from jax.experimental.pallas import tpu_sc as plsc

_B = 16384
_E = 25
_VOCAB = 26
_PAD_ROW = 26
_VP = 32
_NC = 2
_NS = 16
_NW = _NC * _NS
_BPW = _B // _NW
_L = 16

_GDN = lax.GatherDimensionNumbers(
    offset_dims=(), collapsed_slice_dims=(0,), start_index_map=(0,)
)


def _vreg_gather(vec, idx):
    return lax.gather(
        vec, idx[:, None], _GDN, (1,),
        mode=lax.GatherScatterMode.PROMISE_IN_BOUNDS,
    )


@functools.lru_cache(maxsize=1)
def _build():
    mesh = plsc.VectorSubcoreMesh(
        core_axis_name="c", subcore_axis_name="s",
        num_cores=_NC, num_subcores=_NS,
    )

    @functools.partial(
        pl.kernel,
        out_type=jax.ShapeDtypeStruct((_E, _B), jnp.float32),
        mesh=mesh,
        scratch_types=[
            pltpu.VMEM((_E, _VP), jnp.float32),
            pltpu.VMEM((_BPW,), jnp.int32),
            pltpu.VMEM((_BPW,), jnp.int32),
            pltpu.VMEM((_E, _BPW), jnp.float32),
            pltpu.SemaphoreType.DMA,
        ],
        compiler_params=pltpu.CompilerParams(needs_layout_passes=False),
    )
    def emb(wt_hbm, col_hbm, len_hbm, out_hbm, tab_v, col_v, len_v, outt_v, sem):
        wid = lax.axis_index("s") * _NC + lax.axis_index("c")
        base = wid * _BPW
        cps = [
            pltpu.async_copy(wt_hbm, tab_v, sem),
            pltpu.async_copy(col_hbm.at[pl.ds(base, _BPW)], col_v, sem),
            pltpu.async_copy(len_hbm.at[pl.ds(base, _BPW)], len_v, sem),
        ]
        for cp in cps:
            cp.wait()

        @plsc.parallel_loop(0, _BPW, _L, unroll=4)
        def _(off):
            tok = col_v[pl.ds(off, _L)]
            ln = len_v[pl.ds(off, _L)]
            idx = jnp.where(ln > 0, tok, _PAD_ROW)
            lo = idx < _L
            idxm = lax.bitwise_and(idx, _L - 1)
            for c in range(_E):
                va = _vreg_gather(tab_v[c, pl.ds(0, _L)], idxm)
                vb = _vreg_gather(tab_v[c, pl.ds(_L, _L)], idxm)
                outt_v[c, pl.ds(off, _L)] = jnp.where(lo, va, vb)

        pltpu.sync_copy(outt_v, out_hbm.at[:, pl.ds(base, _BPW)])

    return emb


def kernel(input, lengths, weight):
    col = input[:, 0]
    wt = jnp.zeros((_E, _VP), jnp.float32).at[:, :_VOCAB].set(weight.T)
    outt = _build()(wt, col, lengths)
    return outt.T
